# Initial kernel scaffold; baseline (speedup 1.0000x reference)
import functools

import jax
import jax.numpy as jnp
from jax import lax
from jax.experimental import pallas as pl
from jax.experimental.pallas import tpu as pltpu
from jax.experimental.pallas import tpu_sc as plsc

N_NODES = 10000
N_EDGES = 320000
D = 128

NC = 2
NS = 16
NW = NC * NS
NP = 10240
RPT = NP // NS

EPT = N_EDGES // NW
C = 80
NCHUNK = EPT // C
DW = 16


def _sc_agg_body(compute_deg, x_hbm, src_hbm, dst_hbm, zrow_hbm, zdeg_hbm,
                 ones_hbm, agg_out, *rest):
    if compute_deg:
        deg_out, acc_sh, deg_sh, src_v, dst_v, rows_v, ones_v, gsem = rest
    else:
        acc_sh, src_v, dst_v, rows_v, gsem = rest
    c = lax.axis_index("c")
    s = lax.axis_index("s")
    w = s * NC + c

    pltpu.sync_copy(zrow_hbm, acc_sh.at[pl.ds(s * RPT, RPT)])
    if compute_deg:
        pltpu.sync_copy(zdeg_hbm, deg_sh.at[pl.ds(s * RPT, RPT)])
        pltpu.sync_copy(ones_hbm, ones_v)
    plsc.subcore_barrier()

    def load_fire(k, b):
        base = w * EPT + k * C
        pltpu.sync_copy(src_hbm.at[pl.ds(base, C)], src_v.at[b])
        pltpu.sync_copy(dst_hbm.at[pl.ds(base, C)], dst_v.at[b])
        pltpu.async_copy(x_hbm.at[src_v.at[b]], rows_v.at[b], gsem.at[b])

    def process(b):
        pltpu.make_async_copy(x_hbm.at[src_v.at[b]], rows_v.at[b],
                              gsem.at[b]).wait()
        pltpu.sync_copy(rows_v.at[b], acc_sh.at[dst_v.at[b]], add=True)
        if compute_deg:
            pltpu.sync_copy(ones_v, deg_sh.at[dst_v.at[b]], add=True)

    load_fire(0, 0)
    load_fire(1, 1)

    def step(k2, _):
        for b in (0, 1):
            k = k2 * 2 + b
            process(b)
            nk = k + 2

            @pl.when(nk < NCHUNK)
            def _():
                load_fire(nk, b)
        return 0

    lax.fori_loop(0, (NCHUNK - 1) // 2, step, 0)
    process(0)

    plsc.subcore_barrier()
    pltpu.sync_copy(acc_sh.at[pl.ds(s * RPT, RPT)],
                    agg_out.at[c, pl.ds(s * RPT, RPT)])
    if compute_deg:
        pltpu.sync_copy(deg_sh.at[pl.ds(s * RPT, RPT)],
                        deg_out.at[c, pl.ds(s * RPT, RPT)])


def _make_sc_agg(compute_deg):
    mesh = plsc.VectorSubcoreMesh(core_axis_name="c", subcore_axis_name="s",
                                  num_cores=NC, num_subcores=NS)
    out_type = [jax.ShapeDtypeStruct((NC, NP, D), jnp.float32)]
    scratch = [pltpu.VMEM_SHARED((NP, D), jnp.float32)]
    if compute_deg:
        out_type.append(jax.ShapeDtypeStruct((NC, NP, DW), jnp.float32))
        scratch.append(pltpu.VMEM_SHARED((NP, DW), jnp.float32))
    scratch += [
        pltpu.VMEM((2, C), jnp.int32),
        pltpu.VMEM((2, C), jnp.int32),
        pltpu.VMEM((2, C, D), jnp.float32),
    ]
    if compute_deg:
        scratch.append(pltpu.VMEM((C, DW), jnp.float32))
    scratch.append(pltpu.SemaphoreType.DMA((2,)))
    return pl.kernel(functools.partial(_sc_agg_body, compute_deg),
                     out_type=tuple(out_type), mesh=mesh,
                     scratch_types=tuple(scratch))


_sc_agg_deg = _make_sc_agg(True)
_sc_agg = _make_sc_agg(False)


def _dense_body(relu, agg_ref, deg_ref, x_ref, wl_ref, wr_ref, b_ref, o_ref):
    agg = agg_ref[0] + agg_ref[1]
    deg = deg_ref[0, :, 0:1] + deg_ref[1, :, 0:1]
    mean = agg / jnp.maximum(deg, 1.0)
    r = (jnp.dot(mean, wl_ref[...], preferred_element_type=jnp.float32)
         + jnp.dot(x_ref[...], wr_ref[...], preferred_element_type=jnp.float32)
         + b_ref[...])
    o_ref[...] = jnp.maximum(r, 0.0) if relu else r


def _dense(agg, deg, x, w_l, w_r, b, relu):
    return pl.pallas_call(
        functools.partial(_dense_body, relu),
        out_shape=jax.ShapeDtypeStruct((NP, D), jnp.float32),
    )(agg, deg, x, w_l, w_r, b.reshape(1, D))


def kernel(x, edge_index, W1_l, W1_r, b1, W2_l, W2_r, b2):
    src = edge_index[0]
    dst = edge_index[1]
    zrow = jnp.zeros((RPT, D), jnp.float32)
    zdeg = jnp.zeros((RPT, DW), jnp.float32)
    ones = jnp.ones((C, DW), jnp.float32)

    agg1, deg = _sc_agg_deg(x, src, dst, zrow, zdeg, ones)
    x_pad = jnp.zeros((NP, D), jnp.float32).at[:N_NODES].set(x)
    h = _dense(agg1, deg, x_pad, W1_l, W1_r, b1, relu=True)
    (agg2,) = _sc_agg(h, src, dst, zrow, zdeg, ones)
    out = _dense(agg2, deg, h, W2_l, W2_r, b2, relu=False)
    return out[:N_NODES]


# speedup vs baseline: 5.0924x; 5.0924x over previous
import functools

import jax
import jax.numpy as jnp
from jax import lax
from jax.experimental import pallas as pl
from jax.experimental.pallas import tpu as pltpu
from jax.experimental.pallas import tpu_sc as plsc

N_NODES = 10000
N_EDGES = 320000
D = 128

NC = 2
NS = 16
NW = NC * NS
NP = 10240
RPT = NP // NS
ND = NP // D

EPT = N_EDGES // NW
C = 80
NCHUNK = EPT // C
NV = C // 16


def _sc_agg_body(compute_deg, x_hbm, src_hbm, dst_hbm, zrow_hbm,
                 agg_out, *rest):
    if compute_deg:
        deg_out, acc_sh, deg_sh, src_v, dst_v, rows_v, oh_v, row_v, gsem = rest
    else:
        acc_sh, src_v, dst_v, rows_v, gsem = rest
    c = lax.axis_index("c")
    s = lax.axis_index("s")
    w = s * NC + c

    pltpu.sync_copy(zrow_hbm, acc_sh.at[pl.ds(s * RPT, RPT)])
    if compute_deg:
        @pl.when(s < ND // 16)
        def _():
            pltpu.sync_copy(zrow_hbm.at[pl.ds(0, 16)],
                            deg_sh.at[pl.ds(s * 16, 16)])
        pltpu.sync_copy(zrow_hbm.at[pl.ds(0, C)], oh_v)
    plsc.subcore_barrier()

    ones16 = jnp.full((16,), 1.0, jnp.float32)
    zeros16 = jnp.full((16,), 0.0, jnp.float32)

    def step(k, _):
        base = w * EPT + k * C
        pltpu.sync_copy(src_hbm.at[pl.ds(base, C)], src_v)
        pltpu.sync_copy(dst_hbm.at[pl.ds(base, C)], dst_v)
        gather = pltpu.async_copy(x_hbm.at[src_v], rows_v, gsem)
        if compute_deg:
            for i in range(NV):
                dv = dst_v[pl.ds(i * 16, 16)]
                col = jnp.bitwise_and(dv, D - 1)
                row_v[pl.ds(i * 16, 16)] = jnp.right_shift(dv, 7)
                eids = lax.iota(jnp.int32, 16) + jnp.int32(i * 16)
                plsc.addupdate_scatter(oh_v, [eids, col], ones16)
        gather.wait()
        pltpu.sync_copy(rows_v, acc_sh.at[dst_v], add=True)
        if compute_deg:
            pltpu.sync_copy(oh_v, deg_sh.at[row_v], add=True)
            for i in range(NV):
                dv = dst_v[pl.ds(i * 16, 16)]
                col = jnp.bitwise_and(dv, D - 1)
                eids = lax.iota(jnp.int32, 16) + jnp.int32(i * 16)
                plsc.store_scatter(oh_v, [eids, col], zeros16)
        return 0

    lax.fori_loop(0, NCHUNK, step, 0)

    plsc.subcore_barrier()
    pltpu.sync_copy(acc_sh.at[pl.ds(s * RPT, RPT)],
                    agg_out.at[pl.ds(c * NP + s * RPT, RPT)])
    if compute_deg:
        @pl.when(s < ND // 16)
        def _():
            pltpu.sync_copy(deg_sh.at[pl.ds(s * 16, 16)],
                            deg_out.at[pl.ds(c * ND + s * 16, 16)])


def _make_sc_agg(compute_deg):
    mesh = plsc.VectorSubcoreMesh(core_axis_name="c", subcore_axis_name="s",
                                  num_cores=NC, num_subcores=NS)
    out_type = [jax.ShapeDtypeStruct((NC * NP, D), jnp.float32)]
    scratch = [pltpu.VMEM_SHARED((NP, D), jnp.float32)]
    if compute_deg:
        out_type.append(jax.ShapeDtypeStruct((NC * ND, D), jnp.float32))
        scratch.append(pltpu.VMEM_SHARED((ND, D), jnp.float32))
    scratch += [
        pltpu.VMEM((C,), jnp.int32),
        pltpu.VMEM((C,), jnp.int32),
        pltpu.VMEM((C, D), jnp.float32),
    ]
    if compute_deg:
        scratch += [
            pltpu.VMEM((C, D), jnp.float32),
            pltpu.VMEM((C,), jnp.int32),
        ]
    scratch.append(pltpu.SemaphoreType.DMA)
    return pl.kernel(functools.partial(_sc_agg_body, compute_deg),
                     out_type=tuple(out_type), mesh=mesh,
                     scratch_types=tuple(scratch),
                     compiler_params=pltpu.CompilerParams(
                         needs_layout_passes=False))


_sc_agg_deg = _make_sc_agg(True)
_sc_agg = _make_sc_agg(False)


def _dense_body(relu, agg_ref, deg_ref, x_ref, wl_ref, wr_ref, b_ref, o_ref):
    agg = agg_ref[0] + agg_ref[1]
    deg = deg_ref[0] + deg_ref[1]
    mean = agg / jnp.maximum(deg, 1.0)
    r = (jnp.dot(mean, wl_ref[...], preferred_element_type=jnp.float32)
         + jnp.dot(x_ref[...], wr_ref[...], preferred_element_type=jnp.float32)
         + b_ref[...])
    o_ref[...] = jnp.maximum(r, 0.0) if relu else r


def _dense(agg, deg, x, w_l, w_r, b, relu):
    return pl.pallas_call(
        functools.partial(_dense_body, relu),
        out_shape=jax.ShapeDtypeStruct((NP, D), jnp.float32),
    )(agg.reshape(NC, NP, D), deg.reshape(NC, NP, 1), x, w_l, w_r,
      b.reshape(1, D))


def kernel(x, edge_index, W1_l, W1_r, b1, W2_l, W2_r, b2):
    src = edge_index[0]
    dst = edge_index[1]
    zrow = jnp.zeros((RPT, D), jnp.float32)
    x_pad = jnp.zeros((NP, D), jnp.float32).at[:N_NODES].set(x)

    agg1, deg = _sc_agg_deg(x_pad, src, dst, zrow)
    h = _dense(agg1, deg, x_pad, W1_l, W1_r, b1, relu=True)
    (agg2,) = _sc_agg(h, src, dst, zrow)
    out = _dense(agg2, deg, h, W2_l, W2_r, b2, relu=False)
    return out[:N_NODES]


# speedup vs baseline: 8.4404x; 1.6575x over previous
import functools

import jax
import jax.numpy as jnp
from jax import lax
from jax.experimental import pallas as pl
from jax.experimental.pallas import tpu as pltpu
from jax.experimental.pallas import tpu_sc as plsc

N_NODES = 10000
N_EDGES = 320000
D = 128

NC = 2
NS = 16
NW = NC * NS
NP = 10240
RPT = NP // NS
ND = NP // D

EPT = N_EDGES // NW
C = 80
NCHUNK = EPT // C
NV = C // 16
NB = 3


def _sc_agg_body(compute_deg, x_hbm, src_hbm, dst_hbm, zrow_hbm,
                 agg_out, *rest):
    if compute_deg:
        (deg_out, acc_sh, deg_sh, src_v, dst_v, rows_v, oh_v, row_v,
         *sems) = rest
    else:
        acc_sh, src_v, dst_v, rows_v, *sems = rest
    gs = sems[:NB]
    ss = sems[NB:2 * NB]
    c = lax.axis_index("c")
    s = lax.axis_index("s")
    w = s * NC + c

    pltpu.sync_copy(zrow_hbm, acc_sh.at[pl.ds(s * RPT, RPT)])
    if compute_deg:
        @pl.when(s < ND // 16)
        def _():
            pltpu.sync_copy(zrow_hbm.at[pl.ds(0, 16)],
                            deg_sh.at[pl.ds(s * 16, 16)])
        pltpu.sync_copy(zrow_hbm.at[pl.ds(0, C)], oh_v)
    plsc.subcore_barrier()

    ones16 = jnp.full((16,), 1.0, jnp.float32)
    zeros16 = jnp.full((16,), 0.0, jnp.float32)

    def idx_load(k, b):
        base = w * EPT + k * C
        pltpu.sync_copy(src_hbm.at[pl.ds(base, C)], src_v.at[b])
        pltpu.sync_copy(dst_hbm.at[pl.ds(base, C)], dst_v.at[b])

    def gather_fire(b):
        pltpu.async_copy(x_hbm.at[src_v.at[b]], rows_v.at[b], gs[b])

    def gather_wait(b):
        pltpu.make_async_copy(x_hbm.at[src_v.at[b]], rows_v.at[b],
                              gs[b]).wait()

    def scatter_fire(b):
        pltpu.async_copy(rows_v.at[b], acc_sh.at[dst_v.at[b]], ss[b],
                         add=True)

    def scatter_wait(b):
        pltpu.make_async_copy(rows_v.at[b], acc_sh.at[dst_v.at[b]],
                              ss[b]).wait()

    def deg_step(b):
        for i in range(NV):
            dv = dst_v[b, pl.ds(i * 16, 16)]
            col = jnp.bitwise_and(dv, D - 1)
            row_v[pl.ds(i * 16, 16)] = jnp.right_shift(dv, 7)
            eids = lax.iota(jnp.int32, 16) + jnp.int32(i * 16)
            plsc.addupdate_scatter(oh_v, [eids, col], ones16)
        pltpu.sync_copy(oh_v, deg_sh.at[row_v], add=True)
        for i in range(NV):
            dv = dst_v[b, pl.ds(i * 16, 16)]
            col = jnp.bitwise_and(dv, D - 1)
            eids = lax.iota(jnp.int32, 16) + jnp.int32(i * 16)
            plsc.store_scatter(oh_v, [eids, col], zeros16)

    def step(k, b, wait_prev, next_k):
        gather_wait(b)
        scatter_fire(b)
        if compute_deg:
            deg_step(b)
        b2 = (b + 2) % NB
        if wait_prev:
            scatter_wait(b2)
        if next_k is not None:
            idx_load(next_k, b2)
            gather_fire(b2)

    idx_load(0, 0)
    gather_fire(0)
    idx_load(1, 1)
    gather_fire(1)

    step(0, 0, False, 2)
    step(1, 1, True, 3)
    step(2, 2, True, 4)

    def loop_body(k3, _):
        k = k3 * 3
        for b in range(NB):
            step(k + b, b, True, k + b + 2)
        return 0

    lax.fori_loop(1, 41, loop_body, 0)

    step(123, 0, True, None)
    step(124, 1, True, None)
    scatter_wait(1)

    plsc.subcore_barrier()
    pltpu.sync_copy(acc_sh.at[pl.ds(s * RPT, RPT)],
                    agg_out.at[pl.ds(c * NP + s * RPT, RPT)])
    if compute_deg:
        @pl.when(s < ND // 16)
        def _():
            pltpu.sync_copy(deg_sh.at[pl.ds(s * 16, 16)],
                            deg_out.at[pl.ds(c * ND + s * 16, 16)])


def _make_sc_agg(compute_deg):
    mesh = plsc.VectorSubcoreMesh(core_axis_name="c", subcore_axis_name="s",
                                  num_cores=NC, num_subcores=NS)
    out_type = [jax.ShapeDtypeStruct((NC * NP, D), jnp.float32)]
    scratch = [pltpu.VMEM_SHARED((NP, D), jnp.float32)]
    if compute_deg:
        out_type.append(jax.ShapeDtypeStruct((NC * ND, D), jnp.float32))
        scratch.append(pltpu.VMEM_SHARED((ND, D), jnp.float32))
    scratch += [
        pltpu.VMEM((NB, C), jnp.int32),
        pltpu.VMEM((NB, C), jnp.int32),
        pltpu.VMEM((NB, C, D), jnp.float32),
    ]
    if compute_deg:
        scratch += [
            pltpu.VMEM((C, D), jnp.float32),
            pltpu.VMEM((C,), jnp.int32),
        ]
    scratch += [pltpu.SemaphoreType.DMA] * (2 * NB)
    return pl.kernel(functools.partial(_sc_agg_body, compute_deg),
                     out_type=tuple(out_type), mesh=mesh,
                     scratch_types=tuple(scratch),
                     compiler_params=pltpu.CompilerParams(
                         needs_layout_passes=False))


_sc_agg_deg = _make_sc_agg(True)
_sc_agg = _make_sc_agg(False)


def _dense_body(relu, agg_ref, deg_ref, x_ref, wl_ref, wr_ref, b_ref, o_ref):
    agg = agg_ref[0] + agg_ref[1]
    deg = deg_ref[0] + deg_ref[1]
    mean = agg / jnp.maximum(deg, 1.0)
    r = (jnp.dot(mean, wl_ref[...], preferred_element_type=jnp.float32)
         + jnp.dot(x_ref[...], wr_ref[...], preferred_element_type=jnp.float32)
         + b_ref[...])
    o_ref[...] = jnp.maximum(r, 0.0) if relu else r


def _dense(agg, deg, x, w_l, w_r, b, relu):
    return pl.pallas_call(
        functools.partial(_dense_body, relu),
        out_shape=jax.ShapeDtypeStruct((NP, D), jnp.float32),
    )(agg.reshape(NC, NP, D), deg.reshape(NC, NP, 1), x, w_l, w_r,
      b.reshape(1, D))


def kernel(x, edge_index, W1_l, W1_r, b1, W2_l, W2_r, b2):
    src = edge_index[0]
    dst = edge_index[1]
    zrow = jnp.zeros((RPT, D), jnp.float32)
    x_pad = jnp.zeros((NP, D), jnp.float32).at[:N_NODES].set(x)

    agg1, deg = _sc_agg_deg(x_pad, src, dst, zrow)
    h = _dense(agg1, deg, x_pad, W1_l, W1_r, b1, relu=True)
    (agg2,) = _sc_agg(h, src, dst, zrow)
    out = _dense(agg2, deg, h, W2_l, W2_r, b2, relu=False)
    return out[:N_NODES]


# speedup vs baseline: 8.5684x; 1.0152x over previous
import functools

import jax
import jax.numpy as jnp
from jax import lax
from jax.experimental import pallas as pl
from jax.experimental.pallas import tpu as pltpu
from jax.experimental.pallas import tpu_sc as plsc

N_NODES = 10000
N_EDGES = 320000
D = 128

NC = 2
NS = 16
NW = NC * NS
NP = 10240
RPT = NP // NS
ND = NP // D

EPT = N_EDGES // NW
C = 80
NCHUNK = EPT // C
NV = C // 16


def _sc_agg_body(compute_deg, nb, x_hbm, src_hbm, dst_hbm, zrow_hbm,
                 agg_out, *rest):
    if compute_deg:
        (deg_out, acc_sh, deg_sh, src_v, dst_v, rows_v, oh_v, row_v,
         *sems) = rest
    else:
        acc_sh, src_v, dst_v, rows_v, *sems = rest
    gs = sems[:nb]
    ss = sems[nb:2 * nb]
    c = lax.axis_index("c")
    s = lax.axis_index("s")
    w = s * NC + c

    pltpu.sync_copy(zrow_hbm, acc_sh.at[pl.ds(s * RPT, RPT)])
    if compute_deg:
        @pl.when(s < ND // 16)
        def _():
            pltpu.sync_copy(zrow_hbm.at[pl.ds(0, 16)],
                            deg_sh.at[pl.ds(s * 16, 16)])
        pltpu.sync_copy(zrow_hbm.at[pl.ds(0, C)], oh_v)
    plsc.subcore_barrier()

    ones16 = jnp.full((16,), 1.0, jnp.float32)
    zeros16 = jnp.full((16,), 0.0, jnp.float32)

    def idx_load(k, b):
        base = w * EPT + k * C
        pltpu.sync_copy(src_hbm.at[pl.ds(base, C)], src_v.at[b])
        pltpu.sync_copy(dst_hbm.at[pl.ds(base, C)], dst_v.at[b])

    def gather_fire(b):
        pltpu.async_copy(x_hbm.at[src_v.at[b]], rows_v.at[b], gs[b])

    def gather_wait(b):
        pltpu.make_async_copy(x_hbm.at[src_v.at[b]], rows_v.at[b],
                              gs[b]).wait()

    def scatter_fire(b):
        pltpu.async_copy(rows_v.at[b], acc_sh.at[dst_v.at[b]], ss[b],
                         add=True)

    def scatter_wait(b):
        pltpu.make_async_copy(rows_v.at[b], acc_sh.at[dst_v.at[b]],
                              ss[b]).wait()

    def deg_step(b):
        for i in range(NV):
            dv = dst_v[b, pl.ds(i * 16, 16)]
            col = jnp.bitwise_and(dv, D - 1)
            row_v[pl.ds(i * 16, 16)] = jnp.right_shift(dv, 7)
            eids = lax.iota(jnp.int32, 16) + jnp.int32(i * 16)
            plsc.addupdate_scatter(oh_v, [eids, col], ones16)
        pltpu.sync_copy(oh_v, deg_sh.at[row_v], add=True)
        for i in range(NV):
            dv = dst_v[b, pl.ds(i * 16, 16)]
            col = jnp.bitwise_and(dv, D - 1)
            eids = lax.iota(jnp.int32, 16) + jnp.int32(i * 16)
            plsc.store_scatter(oh_v, [eids, col], zeros16)

    def step(k, b, wait_prev, next_k):
        gather_wait(b)
        scatter_fire(b)
        if compute_deg:
            deg_step(b)
        b2 = (b + 2) % nb
        if wait_prev:
            scatter_wait(b2)
        if next_k is not None:
            idx_load(next_k, b2)
            gather_fire(b2)

    idx_load(0, 0)
    gather_fire(0)
    idx_load(1, 1)
    gather_fire(1)

    for k in range(nb):
        step(k, k, k >= nb - 2, k + 2)

    tail_start = (NCHUNK - 2) // nb * nb
    def loop_body(g, _):
        k = g * nb
        for b in range(nb):
            step(k + b, b, True, k + b + 2)
        return 0

    lax.fori_loop(1, tail_start // nb, loop_body, 0)

    for k in range(tail_start, NCHUNK):
        nk = k + 2 if k + 2 < NCHUNK else None
        step(k, k % nb, True, nk)
    for k in range(NCHUNK - (nb - 2), NCHUNK):
        scatter_wait(k % nb)

    plsc.subcore_barrier()
    pltpu.sync_copy(acc_sh.at[pl.ds(s * RPT, RPT)],
                    agg_out.at[pl.ds(c * NP + s * RPT, RPT)])
    if compute_deg:
        @pl.when(s < ND // 16)
        def _():
            pltpu.sync_copy(deg_sh.at[pl.ds(s * 16, 16)],
                            deg_out.at[pl.ds(c * ND + s * 16, 16)])


def _make_sc_agg(compute_deg, nb):
    mesh = plsc.VectorSubcoreMesh(core_axis_name="c", subcore_axis_name="s",
                                  num_cores=NC, num_subcores=NS)
    out_type = [jax.ShapeDtypeStruct((NC * NP, D), jnp.float32)]
    scratch = [pltpu.VMEM_SHARED((NP, D), jnp.float32)]
    if compute_deg:
        out_type.append(jax.ShapeDtypeStruct((NC * ND, D), jnp.float32))
        scratch.append(pltpu.VMEM_SHARED((ND, D), jnp.float32))
    scratch += [
        pltpu.VMEM((nb, C), jnp.int32),
        pltpu.VMEM((nb, C), jnp.int32),
        pltpu.VMEM((nb, C, D), jnp.float32),
    ]
    if compute_deg:
        scratch += [
            pltpu.VMEM((C, D), jnp.float32),
            pltpu.VMEM((C,), jnp.int32),
        ]
    scratch += [pltpu.SemaphoreType.DMA] * (2 * nb)
    return pl.kernel(functools.partial(_sc_agg_body, compute_deg, nb),
                     out_type=tuple(out_type), mesh=mesh,
                     scratch_types=tuple(scratch),
                     compiler_params=pltpu.CompilerParams(
                         needs_layout_passes=False))


_sc_agg_deg = _make_sc_agg(True, 3)
_sc_agg = _make_sc_agg(False, 4)


def _dense_body(relu, agg_ref, deg_ref, x_ref, wl_ref, wr_ref, b_ref, o_ref):
    agg = agg_ref[0, :N_NODES] + agg_ref[1, :N_NODES]
    deg = deg_ref[0, :N_NODES] + deg_ref[1, :N_NODES]
    mean = agg / jnp.maximum(deg, 1.0)
    r = (jnp.dot(mean, wl_ref[...], preferred_element_type=jnp.float32)
         + jnp.dot(x_ref[...], wr_ref[...], preferred_element_type=jnp.float32)
         + b_ref[...])
    o_ref[...] = jnp.maximum(r, 0.0) if relu else r


def _dense(agg, deg, x, w_l, w_r, b, relu):
    return pl.pallas_call(
        functools.partial(_dense_body, relu),
        out_shape=jax.ShapeDtypeStruct((N_NODES, D), jnp.float32),
    )(agg.reshape(NC, NP, D), deg.reshape(NC, NP, 1), x, w_l, w_r,
      b.reshape(1, D))


def kernel(x, edge_index, W1_l, W1_r, b1, W2_l, W2_r, b2):
    src = edge_index[0]
    dst = edge_index[1]
    zrow = jnp.zeros((RPT, D), jnp.float32)

    agg1, deg = _sc_agg_deg(x, src, dst, zrow)
    h = _dense(agg1, deg, x, W1_l, W1_r, b1, relu=True)
    (agg2,) = _sc_agg(h, src, dst, zrow)
    return _dense(agg2, deg, h, W2_l, W2_r, b2, relu=False)


# speedup vs baseline: 13.9594x; 1.6292x over previous
import functools

import jax
import jax.numpy as jnp
from jax import lax
from jax.experimental import pallas as pl
from jax.experimental.pallas import tpu as pltpu
from jax.experimental.pallas import tpu_sc as plsc

N_NODES = 10000
N_EDGES = 320000
D = 128

NC = 2
NS = 16
NW = NC * NS
NP = 10240
RPT = NP // NS
ND = NP // D

EPT = N_EDGES // NW
C = 80
NCHUNK = EPT // C
NV = C // 16


def _sc_agg_body(compute_deg, nb, x_hbm, src_hbm, dst_hbm, zrow_hbm,
                 agg_out, *rest):
    nbi = 2 * nb
    if compute_deg:
        (deg_out, acc_sh, deg_sh, src_v, dst_v, rows_v, ldeg, idx80,
         *sems) = rest
    else:
        acc_sh, src_v, dst_v, rows_v, *sems = rest
    gs = sems[:nb]
    ss = sems[nb:2 * nb]
    isem = sems[2 * nb:2 * nb + nbi]
    c = lax.axis_index("c")
    s = lax.axis_index("s")
    w = s * NC + c

    pltpu.sync_copy(zrow_hbm, acc_sh.at[pl.ds(s * RPT, RPT)])
    if compute_deg:
        @pl.when(s < ND // 16)
        def _():
            pltpu.sync_copy(zrow_hbm.at[pl.ds(0, 16)],
                            deg_sh.at[pl.ds(s * 16, 16)])
        pltpu.sync_copy(zrow_hbm.at[pl.ds(0, ND)], ldeg)
        for i in range(ND // 16):
            idx80[pl.ds(i * 16, 16)] = lax.iota(jnp.int32, 16) + jnp.int32(
                i * 16)
    plsc.subcore_barrier()

    def idx_fire(k, j):
        base = w * EPT + k * C
        pltpu.async_copy(src_hbm.at[pl.ds(base, C)], src_v.at[j], isem[j])
        pltpu.async_copy(dst_hbm.at[pl.ds(base, C)], dst_v.at[j], isem[j])

    def idx_wait(k, j):
        base = w * EPT + k * C
        pltpu.make_async_copy(src_hbm.at[pl.ds(base, C)], src_v.at[j],
                              isem[j]).wait()
        pltpu.make_async_copy(dst_hbm.at[pl.ds(base, C)], dst_v.at[j],
                              isem[j]).wait()

    def gather_fire(b, j):
        pltpu.async_copy(x_hbm.at[src_v.at[j]], rows_v.at[b], gs[b])

    def gather_wait(b, j):
        pltpu.make_async_copy(x_hbm.at[src_v.at[j]], rows_v.at[b],
                              gs[b]).wait()

    def scatter_fire(b, j):
        pltpu.async_copy(rows_v.at[b], acc_sh.at[dst_v.at[j]], ss[b],
                         add=True)

    def scatter_wait(b, j):
        pltpu.make_async_copy(rows_v.at[b], acc_sh.at[dst_v.at[j]],
                              ss[b]).wait()

    iota16 = lax.iota(jnp.int32, 16)

    def deg_step(j):
        for i in range(NV):
            dv = dst_v[j, pl.ds(i * 16, 16)]
            sv, _ = plsc.sort_key_val(dv, dv)
            prv = sv.at[jnp.maximum(iota16 - 1, 0)].get(
                mode='promise_in_bounds')
            nxt = sv.at[jnp.minimum(iota16 + 1, 15)].get(
                mode='promise_in_bounds')
            is_start = (iota16 == 0) | (sv != prv)
            is_end = (iota16 == 15) | (sv != nxt)
            sp = plsc.cummax(jnp.where(is_start, iota16, 0))
            cnt = (iota16 - sp + 1).astype(jnp.float32)
            plsc.addupdate_scatter(
                ldeg, [jnp.right_shift(sv, 7), jnp.bitwise_and(sv, D - 1)],
                cnt, mask=is_end)

    def step(k, pos, wait_prev, fire3, next2):
        b = pos % nb
        j = pos % nbi
        gather_wait(b, j)
        scatter_fire(b, j)
        if compute_deg:
            deg_step(j)
        if wait_prev:
            scatter_wait((pos + 2) % nb, (pos + 2 - nb) % nbi)
        if fire3:
            idx_fire(k + 3, (pos + 3) % nbi)
        if next2:
            idx_wait(k + 2, (pos + 2) % nbi)
            gather_fire((pos + 2) % nb, (pos + 2) % nbi)

    idx_fire(0, 0)
    idx_fire(1, 1)
    idx_fire(2, 2)
    idx_wait(0, 0)
    gather_fire(0, 0)
    idx_wait(1, 1)
    gather_fire(1, 1)

    U = nbi
    for k in range(U):
        step(k, k, k >= nb - 2, True, True)

    tail_start = (NCHUNK - 4) // U * U

    def loop_body(g, _):
        k0 = g * U
        for pos in range(U):
            step(k0 + pos, pos, True, True, True)
        return 0

    lax.fori_loop(1, tail_start // U, loop_body, 0)

    for k in range(tail_start, NCHUNK):
        step(k, k % U, True, k + 3 < NCHUNK, k + 2 < NCHUNK)
    for k in range(NCHUNK - (nb - 2), NCHUNK):
        scatter_wait(k % nb, k % nbi)

    if compute_deg:
        pltpu.sync_copy(ldeg, deg_sh.at[idx80], add=True)
    plsc.subcore_barrier()
    pltpu.sync_copy(acc_sh.at[pl.ds(s * RPT, RPT)],
                    agg_out.at[pl.ds(c * NP + s * RPT, RPT)])
    if compute_deg:
        @pl.when(s < ND // 16)
        def _():
            pltpu.sync_copy(deg_sh.at[pl.ds(s * 16, 16)],
                            deg_out.at[pl.ds(c * ND + s * 16, 16)])


def _make_sc_agg(compute_deg, nb):
    mesh = plsc.VectorSubcoreMesh(core_axis_name="c", subcore_axis_name="s",
                                  num_cores=NC, num_subcores=NS)
    nbi = 2 * nb
    out_type = [jax.ShapeDtypeStruct((NC * NP, D), jnp.float32)]
    scratch = [pltpu.VMEM_SHARED((NP, D), jnp.float32)]
    if compute_deg:
        out_type.append(jax.ShapeDtypeStruct((NC * ND, D), jnp.float32))
        scratch.append(pltpu.VMEM_SHARED((ND, D), jnp.float32))
    scratch += [
        pltpu.VMEM((nbi, C), jnp.int32),
        pltpu.VMEM((nbi, C), jnp.int32),
        pltpu.VMEM((nb, C, D), jnp.float32),
    ]
    if compute_deg:
        scratch += [
            pltpu.VMEM((ND, D), jnp.float32),
            pltpu.VMEM((ND,), jnp.int32),
        ]
    scratch += [pltpu.SemaphoreType.DMA] * (2 * nb + nbi)
    return pl.kernel(functools.partial(_sc_agg_body, compute_deg, nb),
                     out_type=tuple(out_type), mesh=mesh,
                     scratch_types=tuple(scratch),
                     compiler_params=pltpu.CompilerParams(
                         needs_layout_passes=False))


_sc_agg_deg = _make_sc_agg(True, 3)
_sc_agg = _make_sc_agg(False, 4)


def _dense_body(relu, agg_ref, deg_ref, x_ref, wl_ref, wr_ref, b_ref, o_ref):
    agg = agg_ref[0, :N_NODES] + agg_ref[1, :N_NODES]
    deg = deg_ref[0, :N_NODES] + deg_ref[1, :N_NODES]
    mean = agg / jnp.maximum(deg, 1.0)
    r = (jnp.dot(mean, wl_ref[...], preferred_element_type=jnp.float32)
         + jnp.dot(x_ref[...], wr_ref[...], preferred_element_type=jnp.float32)
         + b_ref[...])
    o_ref[...] = jnp.maximum(r, 0.0) if relu else r


def _dense(agg, deg, x, w_l, w_r, b, relu):
    return pl.pallas_call(
        functools.partial(_dense_body, relu),
        out_shape=jax.ShapeDtypeStruct((N_NODES, D), jnp.float32),
    )(agg.reshape(NC, NP, D), deg.reshape(NC, NP, 1), x, w_l, w_r,
      b.reshape(1, D))


def kernel(x, edge_index, W1_l, W1_r, b1, W2_l, W2_r, b2):
    src = edge_index[0]
    dst = edge_index[1]
    zrow = jnp.zeros((RPT, D), jnp.float32)

    agg1, deg = _sc_agg_deg(x, src, dst, zrow)
    h = _dense(agg1, deg, x, W1_l, W1_r, b1, relu=True)
    (agg2,) = _sc_agg(h, src, dst, zrow)
    return _dense(agg2, deg, h, W2_l, W2_r, b2, relu=False)


# speedup vs baseline: 14.9162x; 1.0685x over previous
import functools

import jax
import jax.numpy as jnp
from jax import lax
from jax.experimental import pallas as pl
from jax.experimental.pallas import tpu as pltpu
from jax.experimental.pallas import tpu_sc as plsc

N_NODES = 10000
N_EDGES = 320000
D = 128

NC = 2
NS = 16
NW = NC * NS
NP = 10240
RPT = NP // NS
ND = NP // D

EPT = N_EDGES // NW
C = 80
NCHUNK = EPT // C
NV = C // 16


def _sc_agg_body(compute_deg, nb, x_hbm, src_hbm, dst_hbm, zrow_hbm,
                 agg_out, *rest):
    nbi = 2 * nb
    if compute_deg:
        (deg_out, acc_sh, deg_sh, src_v, dst_v, rows_v, ldeg, idx80,
         *sems) = rest
    else:
        acc_sh, src_v, dst_v, rows_v, *sems = rest
    gs = sems[:nb]
    ss = sems[nb:2 * nb]
    isem = sems[2 * nb:2 * nb + nbi]
    c = lax.axis_index("c")
    s = lax.axis_index("s")
    w = s * NC + c

    pltpu.sync_copy(zrow_hbm, acc_sh.at[pl.ds(s * RPT, RPT)])
    if compute_deg:
        @pl.when(s < ND // 16)
        def _():
            pltpu.sync_copy(zrow_hbm.at[pl.ds(0, 16)],
                            deg_sh.at[pl.ds(s * 16, 16)])
        pltpu.sync_copy(zrow_hbm.at[pl.ds(0, ND)], ldeg)
        for i in range(ND // 16):
            idx80[pl.ds(i * 16, 16)] = lax.iota(jnp.int32, 16) + jnp.int32(
                i * 16)
    plsc.subcore_barrier()

    def idx_fire(k, j):
        base = w * EPT + k * C
        pltpu.async_copy(src_hbm.at[pl.ds(base, C)], src_v.at[j], isem[j])
        pltpu.async_copy(dst_hbm.at[pl.ds(base, C)], dst_v.at[j], isem[j])

    def idx_wait(k, j):
        base = w * EPT + k * C
        pltpu.make_async_copy(src_hbm.at[pl.ds(base, C)], src_v.at[j],
                              isem[j]).wait()
        pltpu.make_async_copy(dst_hbm.at[pl.ds(base, C)], dst_v.at[j],
                              isem[j]).wait()

    def gather_fire(b, j):
        pltpu.async_copy(x_hbm.at[src_v.at[j]], rows_v.at[b], gs[b])

    def gather_wait(b, j):
        pltpu.make_async_copy(x_hbm.at[src_v.at[j]], rows_v.at[b],
                              gs[b]).wait()

    def scatter_fire(b, j):
        pltpu.async_copy(rows_v.at[b], acc_sh.at[dst_v.at[j]], ss[b],
                         add=True)

    def scatter_wait(b, j):
        pltpu.make_async_copy(rows_v.at[b], acc_sh.at[dst_v.at[j]],
                              ss[b]).wait()

    iota16 = lax.iota(jnp.int32, 16)

    def deg_step(j):
        for i in range(NV):
            dv = dst_v[j, pl.ds(i * 16, 16)]
            sv, _ = plsc.sort_key_val(dv, dv)
            prv = sv.at[jnp.maximum(iota16 - 1, 0)].get(
                mode='promise_in_bounds')
            nxt = sv.at[jnp.minimum(iota16 + 1, 15)].get(
                mode='promise_in_bounds')
            is_start = (iota16 == 0) | (sv != prv)
            is_end = (iota16 == 15) | (sv != nxt)
            sp = plsc.cummax(jnp.where(is_start, iota16, 0))
            cnt = (iota16 - sp + 1).astype(jnp.float32)
            plsc.addupdate_scatter(
                ldeg, [jnp.right_shift(sv, 7), jnp.bitwise_and(sv, D - 1)],
                cnt, mask=is_end)

    def step(k, pos, wait_prev, fire3, next2):
        b = pos % nb
        j = pos % nbi
        gather_wait(b, j)
        scatter_fire(b, j)
        if compute_deg:
            deg_step(j)
        if wait_prev:
            scatter_wait((pos + 2) % nb, (pos + 2 - nb) % nbi)
        if fire3:
            idx_fire(k + 3, (pos + 3) % nbi)
        if next2:
            idx_wait(k + 2, (pos + 2) % nbi)
            gather_fire((pos + 2) % nb, (pos + 2) % nbi)

    idx_fire(0, 0)
    idx_fire(1, 1)
    idx_fire(2, 2)
    idx_wait(0, 0)
    gather_fire(0, 0)
    idx_wait(1, 1)
    gather_fire(1, 1)

    U = nbi
    for k in range(U):
        step(k, k, k >= nb - 2, True, True)

    tail_start = (NCHUNK - 4) // U * U

    def loop_body(g, _):
        k0 = g * U
        for pos in range(U):
            step(k0 + pos, pos, True, True, True)
        return 0

    lax.fori_loop(1, tail_start // U, loop_body, 0)

    for k in range(tail_start, NCHUNK):
        step(k, k % U, True, k + 3 < NCHUNK, k + 2 < NCHUNK)
    for k in range(NCHUNK - (nb - 2), NCHUNK):
        scatter_wait(k % nb, k % nbi)

    if compute_deg:
        pltpu.sync_copy(ldeg, deg_sh.at[idx80], add=True)
    plsc.subcore_barrier()
    pltpu.sync_copy(acc_sh.at[pl.ds(s * RPT, RPT)],
                    agg_out.at[pl.ds(c * NP + s * RPT, RPT)])
    if compute_deg:
        @pl.when(s < ND // 16)
        def _():
            pltpu.sync_copy(deg_sh.at[pl.ds(s * 16, 16)],
                            deg_out.at[pl.ds(c * ND + s * 16, 16)])


def _make_sc_agg(compute_deg, nb):
    mesh = plsc.VectorSubcoreMesh(core_axis_name="c", subcore_axis_name="s",
                                  num_cores=NC, num_subcores=NS)
    nbi = 2 * nb
    out_type = [jax.ShapeDtypeStruct((NC * NP, D), jnp.float32)]
    scratch = [pltpu.VMEM_SHARED((NP, D), jnp.float32)]
    if compute_deg:
        out_type.append(jax.ShapeDtypeStruct((NC * ND, D), jnp.float32))
        scratch.append(pltpu.VMEM_SHARED((ND, D), jnp.float32))
    scratch += [
        pltpu.VMEM((nbi, C), jnp.int32),
        pltpu.VMEM((nbi, C), jnp.int32),
        pltpu.VMEM((nb, C, D), jnp.float32),
    ]
    if compute_deg:
        scratch += [
            pltpu.VMEM((ND, D), jnp.float32),
            pltpu.VMEM((ND,), jnp.int32),
        ]
    scratch += [pltpu.SemaphoreType.DMA] * (2 * nb + nbi)
    return pl.kernel(functools.partial(_sc_agg_body, compute_deg, nb),
                     out_type=tuple(out_type), mesh=mesh,
                     scratch_types=tuple(scratch),
                     compiler_params=pltpu.CompilerParams(
                         needs_layout_passes=False))


_sc_agg_deg = _make_sc_agg(True, 3)
_sc_agg = _make_sc_agg(False, 4)


def _dense_body(relu, agg_ref, deg_ref, x_ref, wl_ref, wr_ref, b_ref, o_ref):
    agg = agg_ref[0] + agg_ref[1]
    dinv2 = 1.0 / jnp.maximum(deg_ref[0] + deg_ref[1], 1.0)
    d3 = jnp.broadcast_to(dinv2[:, None, :], (ND, D, D))
    dbc = jnp.swapaxes(d3, 1, 2).reshape(NP, D)
    mean = (agg * dbc)[:N_NODES]
    r = (jnp.dot(mean, wl_ref[...], preferred_element_type=jnp.float32)
         + jnp.dot(x_ref[...], wr_ref[...], preferred_element_type=jnp.float32)
         + b_ref[...])
    o_ref[...] = jnp.maximum(r, 0.0) if relu else r


def _dense(agg, deg, x, w_l, w_r, b, relu):
    return pl.pallas_call(
        functools.partial(_dense_body, relu),
        out_shape=jax.ShapeDtypeStruct((N_NODES, D), jnp.float32),
    )(agg.reshape(NC, NP, D), deg.reshape(NC, ND, D), x, w_l, w_r,
      b.reshape(1, D))


def kernel(x, edge_index, W1_l, W1_r, b1, W2_l, W2_r, b2):
    src = edge_index[0]
    dst = edge_index[1]
    zrow = jnp.zeros((RPT, D), jnp.float32)

    agg1, deg = _sc_agg_deg(x, src, dst, zrow)
    h = _dense(agg1, deg, x, W1_l, W1_r, b1, relu=True)
    (agg2,) = _sc_agg(h, src, dst, zrow)
    return _dense(agg2, deg, h, W2_l, W2_r, b2, relu=False)
